# baseline (speedup 1.0000x reference)
import jax
import jax.numpy as jnp
from jax import lax
from jax.experimental import pallas as pl
from jax.experimental.pallas import tpu as pltpu
from jax.experimental.pallas import tpu_sc as plsc

N = 10000
E = 320000
D = 128
OUT = 64
EPS = 1e-5
FN = float(N)
NPAD = 10240
NC, NS = 2, 16
NW = NC * NS
RPT = NPAD // NS
EPT = E // NW
CH = 80
NCT = EPT // CH
B1 = 1000
GRID = N // B1

_mesh = plsc.VectorSubcoreMesh(core_axis_name="c", subcore_axis_name="s")


def _deg_body(col3_hbm, ones_hbm, zeros_hbm, out_hbm, colv, ones_v, deg_sh):
    c = lax.axis_index("c")
    s = lax.axis_index("s")
    wid = s * NC + c
    pltpu.sync_copy(col3_hbm.at[wid], colv)
    pltpu.sync_copy(ones_hbm, ones_v)
    pltpu.sync_copy(zeros_hbm, deg_sh.at[pl.ds(s * RPT, RPT), :])
    plsc.subcore_barrier()

    def step(k, carry):
        pltpu.sync_copy(ones_v, deg_sh.at[colv.at[k]], add=True)
        return carry

    lax.fori_loop(0, NCT, step, 0)
    plsc.subcore_barrier()
    pltpu.sync_copy(deg_sh.at[pl.ds(s * RPT, RPT), :],
                    out_hbm.at[c, pl.ds(s * RPT, RPT), :])


_deg_call = pl.kernel(
    _deg_body,
    out_type=jax.ShapeDtypeStruct((NC, NPAD, D), jnp.float32),
    mesh=_mesh,
    scratch_types=[
        pltpu.VMEM((NCT, CH), jnp.int32),
        pltpu.VMEM((CH, D), jnp.float32),
        pltpu.VMEM_SHARED((NPAD, D), jnp.float32),
    ],
)


def _agg_body(y_hbm, row3_hbm, col2_hbm, zeros_hbm, out_hbm,
              colv, rowv, buf0, buf1, agg_sh, sem0, sem1):
    c = lax.axis_index("c")
    s = lax.axis_index("s")
    wid = s * NC + c
    pltpu.sync_copy(col2_hbm.at[wid], colv)
    pltpu.sync_copy(row3_hbm.at[wid], rowv)
    pltpu.async_copy(y_hbm.at[colv.at[pl.ds(0, CH)]], buf0, sem0)
    pltpu.sync_copy(zeros_hbm, agg_sh.at[pl.ds(s * RPT, RPT), :])
    plsc.subcore_barrier()

    def step(kk, carry):
        k0 = 2 * kk
        k1 = k0 + 1
        pltpu.async_copy(y_hbm.at[colv.at[pl.ds(k1 * CH, CH)]], buf1, sem1)
        pltpu.make_async_copy(y_hbm.at[colv.at[pl.ds(k0 * CH, CH)]], buf0,
                              sem0).wait()
        pltpu.sync_copy(buf0, agg_sh.at[rowv.at[k0]], add=True)
        pltpu.async_copy(y_hbm.at[colv.at[pl.ds((k1 + 1) * CH, CH)]], buf0,
                         sem0)
        pltpu.make_async_copy(y_hbm.at[colv.at[pl.ds(k1 * CH, CH)]], buf1,
                              sem1).wait()
        pltpu.sync_copy(buf1, agg_sh.at[rowv.at[k1]], add=True)
        return carry

    lax.fori_loop(0, NCT // 2, step, 0)
    pltpu.make_async_copy(y_hbm.at[colv.at[pl.ds((NCT - 1) * CH, CH)]], buf0,
                          sem0).wait()
    pltpu.sync_copy(buf0, agg_sh.at[rowv.at[NCT - 1]], add=True)
    plsc.subcore_barrier()
    pltpu.sync_copy(agg_sh.at[pl.ds(s * RPT, RPT), :],
                    out_hbm.at[c, pl.ds(s * RPT, RPT), :])


_agg_call = pl.kernel(
    _agg_body,
    out_type=jax.ShapeDtypeStruct((NC, NPAD, D), jnp.float32),
    mesh=_mesh,
    scratch_types=[
        pltpu.VMEM((EPT,), jnp.int32),
        pltpu.VMEM((NCT, CH), jnp.int32),
        pltpu.VMEM((CH, D), jnp.float32),
        pltpu.VMEM((CH, D), jnp.float32),
        pltpu.VMEM_SHARED((NPAD, D), jnp.float32),
        pltpu.SemaphoreType.DMA,
        pltpu.SemaphoreType.DMA,
    ],
)


def _p12_body(x_ref, tW0, tb0, ln0g, ln0b, Wq, bq, Wk, bk, Wv, bv,
              gW0, gb0, bn0g, bn0b, dega, degb, ln1g, ln1b,
              g0_o, x1_o, y1_o, d_o,
              h_s, qs_s, vs_s, g0_s, M_s, s_s, sq_s, sk_s):
    p = pl.program_id(0)
    i = pl.program_id(1)

    @pl.when(p == 0)
    def _():
        x = x_ref[...]
        t = jnp.dot(x, tW0[...], preferred_element_type=jnp.float32) + tb0[...]
        mu = jnp.mean(t, axis=-1, keepdims=True)
        var = jnp.mean((t - mu) ** 2, axis=-1, keepdims=True)
        h = jnp.maximum(
            (t - mu) / jnp.sqrt(var + EPS) * ln0g[...] + ln0b[...], 0.0)
        qs = jnp.dot(h, Wq[...], preferred_element_type=jnp.float32) + bq[...]
        ks = jnp.dot(h, Wk[...], preferred_element_type=jnp.float32) + bk[...]
        vs = jnp.dot(h, Wv[...], preferred_element_type=jnp.float32) + bv[...]
        g0 = jnp.dot(x, gW0[...], preferred_element_type=jnp.float32) + gb0[...]
        g0 = jnp.maximum(g0 / jnp.sqrt(1.0 + EPS) * bn0g[...] + bn0b[...], 0.0)
        h_s[pl.ds(i * B1, B1), :] = h
        qs_s[pl.ds(i * B1, B1), :] = qs
        vs_s[pl.ds(i * B1, B1), :] = vs
        g0_s[pl.ds(i * B1, B1), :] = g0
        g0_o[...] = g0

        @pl.when(i == 0)
        def _():
            M_s[...] = jnp.zeros_like(M_s)
            s_s[...] = jnp.zeros_like(s_s)
            sq_s[...] = jnp.zeros_like(sq_s)
            sk_s[...] = jnp.zeros_like(sk_s)

        M_s[...] += lax.dot_general(ks, vs, (((0,), (0,)), ((), ())),
                                    preferred_element_type=jnp.float32)
        s_s[...] += jnp.sum(ks, axis=0, keepdims=True)
        sq_s[...] += jnp.sum(qs * qs)
        sk_s[...] += jnp.sum(ks * ks)

    @pl.when(p == 1)
    def _():
        h = h_s[pl.ds(i * B1, B1), :]
        q = qs_s[pl.ds(i * B1, B1), :]
        vs = vs_s[pl.ds(i * B1, B1), :]
        g0 = g0_s[pl.ds(i * B1, B1), :]
        cc = lax.rsqrt(sq_s[0, 0] * sk_s[0, 0])
        num = jnp.dot(q, M_s[...], preferred_element_type=jnp.float32) * cc \
            + FN * vs
        den = lax.dot_general(q, s_s[...], (((1,), (1,)), ((), ())),
                              preferred_element_type=jnp.float32) * cc + FN
        t = (num / den + h) * 0.5
        mu = jnp.mean(t, axis=-1, keepdims=True)
        var = jnp.mean((t - mu) ** 2, axis=-1, keepdims=True)
        x1_o[...] = jnp.maximum(
            (t - mu) / jnp.sqrt(var + EPS) * ln1g[...] + ln1b[...], 0.0)
        degsum = dega[...] + degb[...]
        dv = jnp.where(degsum > 0.0, lax.rsqrt(degsum), 0.0)
        d_o[...] = dv
        y1_o[...] = dv * g0
        g0_o[...] = g0


def _full(shp):
    return pl.BlockSpec(shp, lambda *_: tuple(0 for _ in shp))


_row = pl.BlockSpec((B1, D), lambda i: (i, 0))
_row1 = pl.BlockSpec((B1, 1), lambda i: (i, 0))
_row_p = pl.BlockSpec((B1, D), lambda p, i: (i, 0))
_row1_p = pl.BlockSpec((B1, 1), lambda p, i: (i, 0))
_w = _full((D, D))
_b = _full((1, D))

_p12_call = pl.pallas_call(
    _p12_body,
    grid=(2, GRID),
    in_specs=[_row_p, _w, _b, _b, _b, _w, _b, _w, _b, _w, _b, _w, _b, _b, _b,
              _row1_p, _row1_p, _b, _b],
    out_specs=[_row_p, _row_p, _row_p, _row1_p],
    out_shape=[
        jax.ShapeDtypeStruct((N, D), jnp.float32),
        jax.ShapeDtypeStruct((N, D), jnp.float32),
        jax.ShapeDtypeStruct((N, D), jnp.float32),
        jax.ShapeDtypeStruct((N, 1), jnp.float32),
    ],
    scratch_shapes=[
        pltpu.VMEM((N, D), jnp.float32),
        pltpu.VMEM((N, D), jnp.float32),
        pltpu.VMEM((N, D), jnp.float32),
        pltpu.VMEM((N, D), jnp.float32),
        pltpu.VMEM((D, D), jnp.float32),
        pltpu.VMEM((1, D), jnp.float32),
        pltpu.VMEM((1, 1), jnp.float32),
        pltpu.VMEM((1, 1), jnp.float32),
    ],
)


def _p3_body(agg_a, agg_b, d, g0, W1, b1, bn1g, bn1b, y2_o):
    agg = (agg_a[...] + agg_b[...]) * d[...]
    t = jnp.dot(agg, W1[...], preferred_element_type=jnp.float32) + b1[...]
    g1 = jnp.maximum(t / jnp.sqrt(1.0 + EPS) * bn1g[...] + bn1b[...], 0.0) + g0[...]
    y2_o[...] = d[...] * g1


_p3_call = pl.pallas_call(
    _p3_body,
    grid=(GRID,),
    in_specs=[_row, _row, _row1, _row, _w, _b, _b, _b],
    out_specs=_row,
    out_shape=jax.ShapeDtypeStruct((N, D), jnp.float32),
)


def _p4_body(agg_a, agg_b, d, g0, x1, W2, b2, bn2g, bn2b, fcW, fcb, out_o):
    agg = (agg_a[...] + agg_b[...]) * d[...]
    t = jnp.dot(agg, W2[...], preferred_element_type=jnp.float32) + b2[...]
    g2 = jnp.maximum(t / jnp.sqrt(1.0 + EPS) * bn2g[...] + bn2b[...], 0.0) + g0[...]
    z = 0.8 * g2 + 0.2 * x1[...]
    out_o[...] = jnp.dot(z, fcW[...], preferred_element_type=jnp.float32) + fcb[...]


_p4_call = pl.pallas_call(
    _p4_body,
    grid=(GRID,),
    in_specs=[_row, _row, _row1, _row, _row, _w, _b, _b, _b,
              _full((D, OUT)), _full((1, OUT))],
    out_specs=pl.BlockSpec((B1, OUT), lambda i: (i, 0)),
    out_shape=jax.ShapeDtypeStruct((N, OUT), jnp.float32),
)


def kernel(x, tW0, tb0, tln0_g, tln0_b, Wq, bq, Wk, bk, Wv, bv, tln1_g, tln1_b,
           gW0, gb0, gbn0_g, gbn0_b, W1, b1, gbn1_g, gbn1_b, W2, b2,
           gbn2_g, gbn2_b, fcW, fcb, edge_index):
    row3 = edge_index[0].astype(jnp.int32).reshape(NW, NCT, CH)
    col3 = edge_index[1].astype(jnp.int32).reshape(NW, NCT, CH)
    col2 = edge_index[1].astype(jnp.int32).reshape(NW, EPT)
    r2 = lambda v: v.reshape(1, -1)

    onesd = jnp.ones((CH, D), jnp.float32)
    zagg = jnp.zeros((RPT, D), jnp.float32)
    degp = _deg_call(col3, onesd, zagg)

    g0, x1, y1, dv = _p12_call(
        x, tW0, r2(tb0), r2(tln0_g), r2(tln0_b), Wq, r2(bq), Wk, r2(bk),
        Wv, r2(bv), gW0, r2(gb0), r2(gbn0_g), r2(gbn0_b),
        degp[0, :N, 0:1], degp[1, :N, 0:1], r2(tln1_g), r2(tln1_b))

    agg1 = _agg_call(y1, row3, col2, zagg)
    y2 = _p3_call(agg1[0, :N], agg1[1, :N], dv, g0, W1, r2(b1),
                  r2(gbn1_g), r2(gbn1_b))
    agg2 = _agg_call(y2, row3, col2, zagg)
    return _p4_call(agg2[0, :N], agg2[1, :N], dv, g0, x1, W2, r2(b2),
                    r2(gbn2_g), r2(gbn2_b), fcW, r2(fcb))


# speedup vs baseline: 1.1666x; 1.1666x over previous
import jax
import jax.numpy as jnp
from jax import lax
from jax.experimental import pallas as pl
from jax.experimental.pallas import tpu as pltpu
from jax.experimental.pallas import tpu_sc as plsc

N = 10000
E = 320000
D = 128
OUT = 64
EPS = 1e-5
FN = float(N)
NPAD = 10240
NC, NS = 2, 16
NW = NC * NS
RPT = NPAD // NS
EPT = E // NW
CH = 80
NCT = EPT // CH
B1 = 1000
GRID = N // B1

_mesh = plsc.VectorSubcoreMesh(core_axis_name="c", subcore_axis_name="s")


def _deg_body(col3_hbm, ones_hbm, zeros_hbm, out_hbm, colv, ones_v, deg_sh):
    c = lax.axis_index("c")
    s = lax.axis_index("s")
    wid = s * NC + c
    pltpu.sync_copy(col3_hbm.at[wid], colv)
    pltpu.sync_copy(ones_hbm, ones_v)
    pltpu.sync_copy(zeros_hbm, deg_sh.at[pl.ds(s * RPT, RPT), :])
    plsc.subcore_barrier()

    def step(k, carry):
        pltpu.sync_copy(ones_v, deg_sh.at[colv.at[k]], add=True)
        return carry

    lax.fori_loop(0, NCT, step, 0)
    plsc.subcore_barrier()
    pltpu.sync_copy(deg_sh.at[pl.ds(s * RPT, RPT), :],
                    out_hbm.at[c, pl.ds(s * RPT, RPT), :])


_deg_call = pl.kernel(
    _deg_body,
    out_type=jax.ShapeDtypeStruct((NC, NPAD, D), jnp.float32),
    mesh=_mesh,
    scratch_types=[
        pltpu.VMEM((NCT, CH), jnp.int32),
        pltpu.VMEM((CH, D), jnp.float32),
        pltpu.VMEM_SHARED((NPAD, D), jnp.float32),
    ],
)


NB = 3


def _agg_body(y_hbm, pk_hbm, zeros_hbm, out_hbm,
              pkv, colb, rowb, bufs, agg_sh, sems):
    c = lax.axis_index("c")
    s = lax.axis_index("s")
    wid = s * NC + c
    pltpu.sync_copy(pk_hbm.at[wid], pkv)

    def unpack(k, slot):
        for j in range(CH // 16):
            v = pkv[pl.ds(k * CH + j * 16, 16)]
            colb[slot, pl.ds(j * 16, 16)] = lax.bitwise_and(v, 0xFFFF)
            rowb[slot, pl.ds(j * 16, 16)] = lax.shift_right_logical(v, 16)

    def issue(k, slot):
        pltpu.async_copy(y_hbm.at[colb.at[slot]],
                         bufs.at[pl.ds(slot * CH, CH), :], sems.at[slot])

    for k in range(NB - 1):
        unpack(k, k)
        issue(k, k)
    pltpu.sync_copy(zeros_hbm, agg_sh.at[pl.ds(s * RPT, RPT), :])
    plsc.subcore_barrier()

    def step(k, carry):
        slot = lax.rem(k, NB)
        nk = k + NB - 1
        nslot = lax.rem(nk, NB)

        @pl.when(nk < NCT)
        def _():
            unpack(nk, nslot)
            issue(nk, nslot)

        pltpu.make_async_copy(y_hbm.at[colb.at[slot]],
                              bufs.at[pl.ds(slot * CH, CH), :],
                              sems.at[slot]).wait()
        pltpu.sync_copy(bufs.at[pl.ds(slot * CH, CH), :],
                        agg_sh.at[rowb.at[slot]], add=True)
        return carry

    lax.fori_loop(0, NCT, step, 0)
    plsc.subcore_barrier()
    pltpu.sync_copy(agg_sh.at[pl.ds(s * RPT, RPT), :],
                    out_hbm.at[c, pl.ds(s * RPT, RPT), :])


_agg_call = pl.kernel(
    _agg_body,
    out_type=jax.ShapeDtypeStruct((NC, NPAD, D), jnp.float32),
    mesh=_mesh,
    scratch_types=[
        pltpu.VMEM((EPT,), jnp.int32),
        pltpu.VMEM((NB, CH), jnp.int32),
        pltpu.VMEM((NB, CH), jnp.int32),
        pltpu.VMEM((NB * CH, D), jnp.float32),
        pltpu.VMEM_SHARED((NPAD, D), jnp.float32),
        pltpu.SemaphoreType.DMA((NB,)),
    ],
)


def _p1_body(x_ref, tW0, tb0, ln0g, ln0b, Wq, bq, Wk, bk, Wv, bv,
             gW0, gb0, bn0g, bn0b,
             h_o, g0_o, qs_o, vs_o, M_o, s_o, sq_o, sk_o):
    i = pl.program_id(0)
    x = x_ref[...]
    t = jnp.dot(x, tW0[...], preferred_element_type=jnp.float32) + tb0[...]
    mu = jnp.mean(t, axis=-1, keepdims=True)
    var = jnp.mean((t - mu) ** 2, axis=-1, keepdims=True)
    h = jnp.maximum((t - mu) / jnp.sqrt(var + EPS) * ln0g[...] + ln0b[...], 0.0)
    h_o[...] = h
    qs = jnp.dot(h, Wq[...], preferred_element_type=jnp.float32) + bq[...]
    ks = jnp.dot(h, Wk[...], preferred_element_type=jnp.float32) + bk[...]
    vs = jnp.dot(h, Wv[...], preferred_element_type=jnp.float32) + bv[...]
    qs_o[...] = qs
    vs_o[...] = vs
    g0 = jnp.dot(x, gW0[...], preferred_element_type=jnp.float32) + gb0[...]
    g0_o[...] = jnp.maximum(g0 / jnp.sqrt(1.0 + EPS) * bn0g[...] + bn0b[...], 0.0)

    @pl.when(i == 0)
    def _():
        M_o[...] = jnp.zeros_like(M_o)
        s_o[...] = jnp.zeros_like(s_o)
        sq_o[...] = jnp.zeros_like(sq_o)
        sk_o[...] = jnp.zeros_like(sk_o)

    M_o[...] += lax.dot_general(ks, vs, (((0,), (0,)), ((), ())),
                                preferred_element_type=jnp.float32)
    s_o[...] += jnp.sum(ks, axis=0, keepdims=True)
    sq_o[...] += jnp.sum(qs * qs)
    sk_o[...] += jnp.sum(ks * ks)


def _full(shp):
    return pl.BlockSpec(shp, lambda *_: tuple(0 for _ in shp))


_row = pl.BlockSpec((B1, D), lambda i: (i, 0))
_row1 = pl.BlockSpec((B1, 1), lambda i: (i, 0))
_w = _full((D, D))
_b = _full((1, D))

_p1_call = pl.pallas_call(
    _p1_body,
    grid=(GRID,),
    in_specs=[_row, _w, _b, _b, _b, _w, _b, _w, _b, _w, _b, _w, _b, _b, _b],
    out_specs=[_row, _row, _row, _row, _w, _b, _full((1, 1)), _full((1, 1))],
    out_shape=[
        jax.ShapeDtypeStruct((N, D), jnp.float32),
        jax.ShapeDtypeStruct((N, D), jnp.float32),
        jax.ShapeDtypeStruct((N, D), jnp.float32),
        jax.ShapeDtypeStruct((N, D), jnp.float32),
        jax.ShapeDtypeStruct((D, D), jnp.float32),
        jax.ShapeDtypeStruct((1, D), jnp.float32),
        jax.ShapeDtypeStruct((1, 1), jnp.float32),
        jax.ShapeDtypeStruct((1, 1), jnp.float32),
    ],
)


def _p2_body(h, qs, vs, g0, dega, degb, M, s, sq, sk, ln1g, ln1b,
             x1_o, y1_o, d_o):
    q = qs[...]
    cc = lax.rsqrt(sq[0, 0] * sk[0, 0])
    num = jnp.dot(q, M[...], preferred_element_type=jnp.float32) * cc + FN * vs[...]
    den = lax.dot_general(q, s[...], (((1,), (1,)), ((), ())),
                          preferred_element_type=jnp.float32) * cc + FN
    t = (num / den + h[...]) * 0.5
    mu = jnp.mean(t, axis=-1, keepdims=True)
    var = jnp.mean((t - mu) ** 2, axis=-1, keepdims=True)
    x1_o[...] = jnp.maximum(
        (t - mu) / jnp.sqrt(var + EPS) * ln1g[...] + ln1b[...], 0.0)
    degsum = dega[...] + degb[...]
    dv = jnp.where(degsum > 0.0, lax.rsqrt(degsum), 0.0)
    d_o[...] = dv
    y1_o[...] = dv * g0[...]


_p2_call = pl.pallas_call(
    _p2_body,
    grid=(GRID,),
    in_specs=[_row, _row, _row, _row, _row1, _row1, _w, _b,
              _full((1, 1)), _full((1, 1)), _b, _b],
    out_specs=[_row, _row, _row1],
    out_shape=[
        jax.ShapeDtypeStruct((N, D), jnp.float32),
        jax.ShapeDtypeStruct((N, D), jnp.float32),
        jax.ShapeDtypeStruct((N, 1), jnp.float32),
    ],
)


def _p3_body(agg_a, agg_b, d, g0, W1, b1, bn1g, bn1b, y2_o):
    agg = (agg_a[...] + agg_b[...]) * d[...]
    t = jnp.dot(agg, W1[...], preferred_element_type=jnp.float32) + b1[...]
    g1 = jnp.maximum(t / jnp.sqrt(1.0 + EPS) * bn1g[...] + bn1b[...], 0.0) + g0[...]
    y2_o[...] = d[...] * g1


_p3_call = pl.pallas_call(
    _p3_body,
    grid=(GRID,),
    in_specs=[_row, _row, _row1, _row, _w, _b, _b, _b],
    out_specs=_row,
    out_shape=jax.ShapeDtypeStruct((N, D), jnp.float32),
)


def _p4_body(agg_a, agg_b, d, g0, x1, W2, b2, bn2g, bn2b, fcW, fcb, out_o):
    agg = (agg_a[...] + agg_b[...]) * d[...]
    t = jnp.dot(agg, W2[...], preferred_element_type=jnp.float32) + b2[...]
    g2 = jnp.maximum(t / jnp.sqrt(1.0 + EPS) * bn2g[...] + bn2b[...], 0.0) + g0[...]
    z = 0.8 * g2 + 0.2 * x1[...]
    out_o[...] = jnp.dot(z, fcW[...], preferred_element_type=jnp.float32) + fcb[...]


_p4_call = pl.pallas_call(
    _p4_body,
    grid=(GRID,),
    in_specs=[_row, _row, _row1, _row, _row, _w, _b, _b, _b,
              _full((D, OUT)), _full((1, OUT))],
    out_specs=pl.BlockSpec((B1, OUT), lambda i: (i, 0)),
    out_shape=jax.ShapeDtypeStruct((N, OUT), jnp.float32),
)


def kernel(x, tW0, tb0, tln0_g, tln0_b, Wq, bq, Wk, bk, Wv, bv, tln1_g, tln1_b,
           gW0, gb0, gbn0_g, gbn0_b, W1, b1, gbn1_g, gbn1_b, W2, b2,
           gbn2_g, gbn2_b, fcW, fcb, edge_index):
    row = edge_index[0].astype(jnp.int32)
    col = edge_index[1].astype(jnp.int32)
    col3 = col.reshape(NW, NCT, CH)
    pk2 = (col | (row << 16)).reshape(NW, EPT)
    r2 = lambda v: v.reshape(1, -1)

    onesd = jnp.ones((CH, D), jnp.float32)
    zagg = jnp.zeros((RPT, D), jnp.float32)
    degp = _deg_call(col3, onesd, zagg)

    h, g0, qs, vs, M, s, sq, sk = _p1_call(
        x, tW0, r2(tb0), r2(tln0_g), r2(tln0_b), Wq, r2(bq), Wk, r2(bk),
        Wv, r2(bv), gW0, r2(gb0), r2(gbn0_g), r2(gbn0_b))

    x1, y1, dv = _p2_call(
        h, qs, vs, g0, degp[0, :N, 0:1], degp[1, :N, 0:1], M, s, sq, sk,
        r2(tln1_g), r2(tln1_b))

    agg1 = _agg_call(y1, pk2, zagg)
    y2 = _p3_call(agg1[0, :N], agg1[1, :N], dv, g0, W1, r2(b1),
                  r2(gbn1_g), r2(gbn1_b))
    agg2 = _agg_call(y2, pk2, zagg)
    return _p4_call(agg2[0, :N], agg2[1, :N], dv, g0, x1, W2, r2(b2),
                    r2(gbn2_g), r2(gbn2_b), fcW, r2(fcb))


# speedup vs baseline: 1.1701x; 1.0030x over previous
import jax
import jax.numpy as jnp
from jax import lax
from jax.experimental import pallas as pl
from jax.experimental.pallas import tpu as pltpu
from jax.experimental.pallas import tpu_sc as plsc

N = 10000
E = 320000
D = 128
OUT = 64
EPS = 1e-5
FN = float(N)
NPAD = 10240
NC, NS = 2, 16
NW = NC * NS
RPT = NPAD // NS
EPT = E // NW
CH = 80
NCT = EPT // CH
B1 = 1000
GRID = N // B1

_mesh = plsc.VectorSubcoreMesh(core_axis_name="c", subcore_axis_name="s")


def _deg_body(col3_hbm, ones_hbm, zeros_hbm, out_hbm, colv, ones_v, deg_sh):
    c = lax.axis_index("c")
    s = lax.axis_index("s")
    wid = s * NC + c
    pltpu.sync_copy(col3_hbm.at[wid], colv)
    pltpu.sync_copy(ones_hbm, ones_v)
    pltpu.sync_copy(zeros_hbm, deg_sh.at[pl.ds(s * RPT, RPT), :])
    plsc.subcore_barrier()

    def step(k, carry):
        pltpu.sync_copy(ones_v, deg_sh.at[colv.at[k]], add=True)
        return carry

    lax.fori_loop(0, NCT, step, 0)
    plsc.subcore_barrier()
    pltpu.sync_copy(deg_sh.at[pl.ds(s * RPT, RPT), :],
                    out_hbm.at[c, pl.ds(s * RPT, RPT), :])


_deg_call = pl.kernel(
    _deg_body,
    out_type=jax.ShapeDtypeStruct((NC, NPAD, D), jnp.float32),
    mesh=_mesh,
    scratch_types=[
        pltpu.VMEM((NCT, CH), jnp.int32),
        pltpu.VMEM((CH, D), jnp.float32),
        pltpu.VMEM_SHARED((NPAD, D), jnp.float32),
    ],
)


NB = 3


def _agg_body(y_hbm, pk_hbm, zeros_hbm, out_hbm,
              pkv, colb, rowb, bufs, agg_sh, sems):
    c = lax.axis_index("c")
    s = lax.axis_index("s")
    wid = s * NC + c
    pltpu.sync_copy(pk_hbm.at[wid], pkv)

    def unpack(k, slot):
        for j in range(CH // 16):
            v = pkv[pl.ds(k * CH + j * 16, 16)]
            colb[slot, pl.ds(j * 16, 16)] = lax.bitwise_and(v, 0xFFFF)
            rowb[slot, pl.ds(j * 16, 16)] = lax.shift_right_logical(v, 16)

    def issue(k, slot):
        pltpu.async_copy(y_hbm.at[colb.at[slot]],
                         bufs.at[pl.ds(slot * CH, CH), :], sems.at[slot])

    for k in range(NB - 1):
        unpack(k, k)
        issue(k, k)
    pltpu.sync_copy(zeros_hbm, agg_sh.at[pl.ds(s * RPT, RPT), :])
    plsc.subcore_barrier()

    def step(k, carry):
        slot = lax.rem(k, NB)
        nk = k + NB - 1
        nslot = lax.rem(nk, NB)

        @pl.when(nk < NCT)
        def _():
            unpack(nk, nslot)
            issue(nk, nslot)

        pltpu.make_async_copy(y_hbm.at[colb.at[slot]],
                              bufs.at[pl.ds(slot * CH, CH), :],
                              sems.at[slot]).wait()
        pltpu.sync_copy(bufs.at[pl.ds(slot * CH, CH), :],
                        agg_sh.at[rowb.at[slot]], add=True)
        return carry

    lax.fori_loop(0, NCT, step, 0)
    plsc.subcore_barrier()
    pltpu.sync_copy(agg_sh.at[pl.ds(s * RPT, RPT), :],
                    out_hbm.at[c, pl.ds(s * RPT, RPT), :])


_agg_call = pl.kernel(
    _agg_body,
    out_type=jax.ShapeDtypeStruct((NC, NPAD, D), jnp.float32),
    mesh=_mesh,
    scratch_types=[
        pltpu.VMEM((EPT,), jnp.int32),
        pltpu.VMEM((NB, CH), jnp.int32),
        pltpu.VMEM((NB, CH), jnp.int32),
        pltpu.VMEM((NB * CH, D), jnp.float32),
        pltpu.VMEM_SHARED((NPAD, D), jnp.float32),
        pltpu.SemaphoreType.DMA((NB,)),
    ],
)


def _p1_body(x_ref, tW0, tb0, ln0g, ln0b, Wq, bq, Wk, bk, Wv, bv,
             gW0, gb0, bn0g, bn0b,
             h_o, g0_o, qs_o, vs_o, M_o, s_o, sq_o, sk_o):
    i = pl.program_id(0)
    x = x_ref[...]
    t = jnp.dot(x, tW0[...], preferred_element_type=jnp.float32) + tb0[...]
    mu = jnp.mean(t, axis=-1, keepdims=True)
    var = jnp.mean((t - mu) ** 2, axis=-1, keepdims=True)
    h = jnp.maximum((t - mu) / jnp.sqrt(var + EPS) * ln0g[...] + ln0b[...], 0.0)
    h_o[...] = h
    qs = jnp.dot(h, Wq[...], preferred_element_type=jnp.float32) + bq[...]
    ks = jnp.dot(h, Wk[...], preferred_element_type=jnp.float32) + bk[...]
    vs = jnp.dot(h, Wv[...], preferred_element_type=jnp.float32) + bv[...]
    qs_o[...] = qs
    vs_o[...] = vs
    g0 = jnp.dot(x, gW0[...], preferred_element_type=jnp.float32) + gb0[...]
    g0_o[...] = jnp.maximum(g0 / jnp.sqrt(1.0 + EPS) * bn0g[...] + bn0b[...], 0.0)

    @pl.when(i == 0)
    def _():
        M_o[...] = jnp.zeros_like(M_o)
        s_o[...] = jnp.zeros_like(s_o)
        sq_o[...] = jnp.zeros_like(sq_o)
        sk_o[...] = jnp.zeros_like(sk_o)

    M_o[...] += lax.dot_general(ks, vs, (((0,), (0,)), ((), ())),
                                preferred_element_type=jnp.float32)
    s_o[...] += jnp.sum(ks, axis=0, keepdims=True)
    sq_o[...] += jnp.sum(qs * qs)
    sk_o[...] += jnp.sum(ks * ks)


def _full(shp):
    return pl.BlockSpec(shp, lambda *_: tuple(0 for _ in shp))


_row = pl.BlockSpec((B1, D), lambda i: (i, 0))
_row1 = pl.BlockSpec((B1, 1), lambda i: (i, 0))
_w = _full((D, D))
_b = _full((1, D))

_p1_call = pl.pallas_call(
    _p1_body,
    grid=(GRID,),
    in_specs=[_row, _w, _b, _b, _b, _w, _b, _w, _b, _w, _b, _w, _b, _b, _b],
    out_specs=[_row, _row, _row, _row, _w, _b, _full((1, 1)), _full((1, 1))],
    out_shape=[
        jax.ShapeDtypeStruct((N, D), jnp.float32),
        jax.ShapeDtypeStruct((N, D), jnp.float32),
        jax.ShapeDtypeStruct((N, D), jnp.float32),
        jax.ShapeDtypeStruct((N, D), jnp.float32),
        jax.ShapeDtypeStruct((D, D), jnp.float32),
        jax.ShapeDtypeStruct((1, D), jnp.float32),
        jax.ShapeDtypeStruct((1, 1), jnp.float32),
        jax.ShapeDtypeStruct((1, 1), jnp.float32),
    ],
)


def _p2_body(h, qs, vs, g0, dega, degb, M, s, sq, sk, ln1g, ln1b,
             x1_o, y1_o, d_o):
    q = qs[...]
    cc = lax.rsqrt(sq[0, 0] * sk[0, 0])
    num = jnp.dot(q, M[...], preferred_element_type=jnp.float32) * cc + FN * vs[...]
    den = lax.dot_general(q, s[...], (((1,), (1,)), ((), ())),
                          preferred_element_type=jnp.float32) * cc + FN
    t = (num / den + h[...]) * 0.5
    mu = jnp.mean(t, axis=-1, keepdims=True)
    var = jnp.mean((t - mu) ** 2, axis=-1, keepdims=True)
    x1_o[...] = jnp.maximum(
        (t - mu) / jnp.sqrt(var + EPS) * ln1g[...] + ln1b[...], 0.0)
    degsum = dega[...] + degb[...]
    dv = jnp.where(degsum > 0.0, lax.rsqrt(degsum), 0.0)
    d_o[...] = dv
    y1_o[...] = dv * g0[...]


_p2_call = pl.pallas_call(
    _p2_body,
    grid=(GRID,),
    in_specs=[_row, _row, _row, _row, _row, _row, _w, _b,
              _full((1, 1)), _full((1, 1)), _b, _b],
    out_specs=[_row, _row, _row],
    out_shape=[
        jax.ShapeDtypeStruct((N, D), jnp.float32),
        jax.ShapeDtypeStruct((N, D), jnp.float32),
        jax.ShapeDtypeStruct((N, D), jnp.float32),
    ],
)


def _p3_body(agg_a, agg_b, d, g0, W1, b1, bn1g, bn1b, y2_o):
    agg = (agg_a[...] + agg_b[...]) * d[...]
    t = jnp.dot(agg, W1[...], preferred_element_type=jnp.float32) + b1[...]
    g1 = jnp.maximum(t / jnp.sqrt(1.0 + EPS) * bn1g[...] + bn1b[...], 0.0) + g0[...]
    y2_o[...] = d[...] * g1


_p3_call = pl.pallas_call(
    _p3_body,
    grid=(GRID,),
    in_specs=[_row, _row, _row, _row, _w, _b, _b, _b],
    out_specs=_row,
    out_shape=jax.ShapeDtypeStruct((N, D), jnp.float32),
)


def _p4_body(agg_a, agg_b, d, g0, x1, W2, b2, bn2g, bn2b, fcW, fcb, out_o):
    agg = (agg_a[...] + agg_b[...]) * d[...]
    t = jnp.dot(agg, W2[...], preferred_element_type=jnp.float32) + b2[...]
    g2 = jnp.maximum(t / jnp.sqrt(1.0 + EPS) * bn2g[...] + bn2b[...], 0.0) + g0[...]
    z = 0.8 * g2 + 0.2 * x1[...]
    out_o[...] = jnp.dot(z, fcW[...], preferred_element_type=jnp.float32) + fcb[...]


_p4_call = pl.pallas_call(
    _p4_body,
    grid=(GRID,),
    in_specs=[_row, _row, _row, _row, _row, _w, _b, _b, _b,
              _full((D, OUT)), _full((1, OUT))],
    out_specs=pl.BlockSpec((B1, OUT), lambda i: (i, 0)),
    out_shape=jax.ShapeDtypeStruct((N, OUT), jnp.float32),
)


def kernel(x, tW0, tb0, tln0_g, tln0_b, Wq, bq, Wk, bk, Wv, bv, tln1_g, tln1_b,
           gW0, gb0, gbn0_g, gbn0_b, W1, b1, gbn1_g, gbn1_b, W2, b2,
           gbn2_g, gbn2_b, fcW, fcb, edge_index):
    row = edge_index[0].astype(jnp.int32)
    col = edge_index[1].astype(jnp.int32)
    col3 = col.reshape(NW, NCT, CH)
    pk2 = (col | (row << 16)).reshape(NW, EPT)
    r2 = lambda v: v.reshape(1, -1)

    onesd = jnp.ones((CH, D), jnp.float32)
    zagg = jnp.zeros((RPT, D), jnp.float32)
    degp = _deg_call(col3, onesd, zagg)

    h, g0, qs, vs, M, s, sq, sk = _p1_call(
        x, tW0, r2(tb0), r2(tln0_g), r2(tln0_b), Wq, r2(bq), Wk, r2(bk),
        Wv, r2(bv), gW0, r2(gb0), r2(gbn0_g), r2(gbn0_b))

    x1, y1, dv = _p2_call(
        h, qs, vs, g0, degp[0, :N], degp[1, :N], M, s, sq, sk,
        r2(tln1_g), r2(tln1_b))

    agg1 = _agg_call(y1, pk2, zagg)
    y2 = _p3_call(agg1[0, :N], agg1[1, :N], dv, g0, W1, r2(b1),
                  r2(gbn1_g), r2(gbn1_b))
    agg2 = _agg_call(y2, pk2, zagg)
    return _p4_call(agg2[0, :N], agg2[1, :N], dv, g0, x1, W2, r2(b2),
                    r2(gbn2_g), r2(gbn2_b), fcW, r2(fcb))


# speedup vs baseline: 1.1711x; 1.0008x over previous
import jax
import jax.numpy as jnp
from jax import lax
from jax.experimental import pallas as pl
from jax.experimental.pallas import tpu as pltpu
from jax.experimental.pallas import tpu_sc as plsc

N = 10000
E = 320000
D = 128
OUT = 64
EPS = 1e-5
FN = float(N)
NPAD = 10240
NC, NS = 2, 16
NW = NC * NS
RPT = NPAD // NS
EPT = E // NW
CH = 80
NCT = EPT // CH
B1 = 1000
GRID = N // B1

_mesh = plsc.VectorSubcoreMesh(core_axis_name="c", subcore_axis_name="s")


def _deg_body(col3_hbm, ones_hbm, zeros_hbm, out_hbm, colv, ones_v, deg_sh):
    c = lax.axis_index("c")
    s = lax.axis_index("s")
    wid = s * NC + c
    pltpu.sync_copy(col3_hbm.at[wid], colv)
    pltpu.sync_copy(ones_hbm, ones_v)
    pltpu.sync_copy(zeros_hbm, deg_sh.at[pl.ds(s * RPT, RPT), :])
    plsc.subcore_barrier()

    def step(k, carry):
        pltpu.sync_copy(ones_v, deg_sh.at[colv.at[k]], add=True)
        return carry

    lax.fori_loop(0, NCT, step, 0)
    plsc.subcore_barrier()
    pltpu.sync_copy(deg_sh.at[pl.ds(s * RPT, RPT), :],
                    out_hbm.at[c, pl.ds(s * RPT, RPT), :])


_deg_call = pl.kernel(
    _deg_body,
    out_type=jax.ShapeDtypeStruct((NC, NPAD, D), jnp.float32),
    mesh=_mesh,
    scratch_types=[
        pltpu.VMEM((NCT, CH), jnp.int32),
        pltpu.VMEM((CH, D), jnp.float32),
        pltpu.VMEM_SHARED((NPAD, D), jnp.float32),
    ],
)


NB = 3


def _agg_body(y_hbm, pk_hbm, zeros_hbm, out_hbm,
              pkv, colb, rowb, bufs, agg_sh, sems):
    c = lax.axis_index("c")
    s = lax.axis_index("s")
    wid = s * NC + c
    pltpu.sync_copy(pk_hbm.at[wid], pkv)

    def unpack(k, slot):
        for j in range(CH // 16):
            v = pkv[pl.ds(k * CH + j * 16, 16)]
            colb[slot, pl.ds(j * 16, 16)] = lax.bitwise_and(v, 0xFFFF)
            rowb[slot, pl.ds(j * 16, 16)] = lax.shift_right_logical(v, 16)

    def issue(k, slot):
        pltpu.async_copy(y_hbm.at[colb.at[slot]],
                         bufs.at[pl.ds(slot * CH, CH), :], sems.at[slot])

    for k in range(NB - 1):
        unpack(k, k)
        issue(k, k)
    pltpu.sync_copy(zeros_hbm, agg_sh.at[pl.ds(s * RPT, RPT), :])
    plsc.subcore_barrier()

    def step(k, carry):
        slot = lax.rem(k, NB)
        nk = k + NB - 1
        nslot = lax.rem(nk, NB)

        @pl.when(nk < NCT)
        def _():
            unpack(nk, nslot)
            issue(nk, nslot)

        pltpu.make_async_copy(y_hbm.at[colb.at[slot]],
                              bufs.at[pl.ds(slot * CH, CH), :],
                              sems.at[slot]).wait()
        pltpu.sync_copy(bufs.at[pl.ds(slot * CH, CH), :],
                        agg_sh.at[rowb.at[slot]], add=True)
        return carry

    lax.fori_loop(0, NCT, step, 0)
    plsc.subcore_barrier()
    pltpu.sync_copy(agg_sh.at[pl.ds(s * RPT, RPT), :],
                    out_hbm.at[c, pl.ds(s * RPT, RPT), :])


_agg_call = pl.kernel(
    _agg_body,
    out_type=jax.ShapeDtypeStruct((NC, NPAD, D), jnp.float32),
    mesh=_mesh,
    scratch_types=[
        pltpu.VMEM((EPT,), jnp.int32),
        pltpu.VMEM((NB, CH), jnp.int32),
        pltpu.VMEM((NB, CH), jnp.int32),
        pltpu.VMEM((NB * CH, D), jnp.float32),
        pltpu.VMEM_SHARED((NPAD, D), jnp.float32),
        pltpu.SemaphoreType.DMA((NB,)),
    ],
)


def _p1_body(x_ref, tW0, tb0, ln0g, ln0b, Wq, bq, Wk, bk, Wv, bv,
             gW0, gb0, bn0g, bn0b,
             g0_o, M_o, s_o, sq_o, sk_o):
    i = pl.program_id(0)
    x = x_ref[...]
    t = jnp.dot(x, tW0[...], preferred_element_type=jnp.float32) + tb0[...]
    mu = jnp.mean(t, axis=-1, keepdims=True)
    var = jnp.mean((t - mu) ** 2, axis=-1, keepdims=True)
    h = jnp.maximum((t - mu) / jnp.sqrt(var + EPS) * ln0g[...] + ln0b[...], 0.0)
    qs = jnp.dot(h, Wq[...], preferred_element_type=jnp.float32) + bq[...]
    ks = jnp.dot(h, Wk[...], preferred_element_type=jnp.float32) + bk[...]
    vs = jnp.dot(h, Wv[...], preferred_element_type=jnp.float32) + bv[...]
    g0 = jnp.dot(x, gW0[...], preferred_element_type=jnp.float32) + gb0[...]
    g0_o[...] = jnp.maximum(g0 / jnp.sqrt(1.0 + EPS) * bn0g[...] + bn0b[...], 0.0)

    @pl.when(i == 0)
    def _():
        M_o[...] = jnp.zeros_like(M_o)
        s_o[...] = jnp.zeros_like(s_o)
        sq_o[...] = jnp.zeros_like(sq_o)
        sk_o[...] = jnp.zeros_like(sk_o)

    M_o[...] += lax.dot_general(ks, vs, (((0,), (0,)), ((), ())),
                                preferred_element_type=jnp.float32)
    s_o[...] += jnp.sum(ks, axis=0, keepdims=True)
    sq_o[...] += jnp.sum(qs * qs)
    sk_o[...] += jnp.sum(ks * ks)


def _full(shp):
    return pl.BlockSpec(shp, lambda *_: tuple(0 for _ in shp))


_row = pl.BlockSpec((B1, D), lambda i: (i, 0))
_row1 = pl.BlockSpec((B1, 1), lambda i: (i, 0))
_w = _full((D, D))
_b = _full((1, D))

_p1_call = pl.pallas_call(
    _p1_body,
    grid=(GRID,),
    in_specs=[_row, _w, _b, _b, _b, _w, _b, _w, _b, _w, _b, _w, _b, _b, _b],
    out_specs=[_row, _w, _b, _full((1, 1)), _full((1, 1))],
    out_shape=[
        jax.ShapeDtypeStruct((N, D), jnp.float32),
        jax.ShapeDtypeStruct((D, D), jnp.float32),
        jax.ShapeDtypeStruct((1, D), jnp.float32),
        jax.ShapeDtypeStruct((1, 1), jnp.float32),
        jax.ShapeDtypeStruct((1, 1), jnp.float32),
    ],
)


def _p2_body(x_ref, tW0, tb0, ln0g, ln0b, Wq, bq, Wv, bv, g0,
             dega, degb, M, s, sq, sk, ln1g, ln1b,
             x1_o, y1_o, d_o):
    x = x_ref[...]
    tt = jnp.dot(x, tW0[...], preferred_element_type=jnp.float32) + tb0[...]
    mu0 = jnp.mean(tt, axis=-1, keepdims=True)
    var0 = jnp.mean((tt - mu0) ** 2, axis=-1, keepdims=True)
    h = jnp.maximum(
        (tt - mu0) / jnp.sqrt(var0 + EPS) * ln0g[...] + ln0b[...], 0.0)
    q = jnp.dot(h, Wq[...], preferred_element_type=jnp.float32) + bq[...]
    vs = jnp.dot(h, Wv[...], preferred_element_type=jnp.float32) + bv[...]
    cc = lax.rsqrt(sq[0, 0] * sk[0, 0])
    num = jnp.dot(q, M[...], preferred_element_type=jnp.float32) * cc + FN * vs
    den = lax.dot_general(q, s[...], (((1,), (1,)), ((), ())),
                          preferred_element_type=jnp.float32) * cc + FN
    t = (num / den + h) * 0.5
    mu = jnp.mean(t, axis=-1, keepdims=True)
    var = jnp.mean((t - mu) ** 2, axis=-1, keepdims=True)
    x1_o[...] = jnp.maximum(
        (t - mu) / jnp.sqrt(var + EPS) * ln1g[...] + ln1b[...], 0.0)
    degsum = dega[...] + degb[...]
    dv = jnp.where(degsum > 0.0, lax.rsqrt(degsum), 0.0)
    d_o[...] = dv
    y1_o[...] = dv * g0[...]


_p2_call = pl.pallas_call(
    _p2_body,
    grid=(GRID,),
    in_specs=[_row, _w, _b, _b, _b, _w, _b, _w, _b, _row, _row, _row,
              _w, _b, _full((1, 1)), _full((1, 1)), _b, _b],
    out_specs=[_row, _row, _row],
    out_shape=[
        jax.ShapeDtypeStruct((N, D), jnp.float32),
        jax.ShapeDtypeStruct((N, D), jnp.float32),
        jax.ShapeDtypeStruct((N, D), jnp.float32),
    ],
)


def _p3_body(agg_a, agg_b, d, g0, W1, b1, bn1g, bn1b, y2_o):
    agg = (agg_a[...] + agg_b[...]) * d[...]
    t = jnp.dot(agg, W1[...], preferred_element_type=jnp.float32) + b1[...]
    g1 = jnp.maximum(t / jnp.sqrt(1.0 + EPS) * bn1g[...] + bn1b[...], 0.0) + g0[...]
    y2_o[...] = d[...] * g1


_p3_call = pl.pallas_call(
    _p3_body,
    grid=(GRID,),
    in_specs=[_row, _row, _row, _row, _w, _b, _b, _b],
    out_specs=_row,
    out_shape=jax.ShapeDtypeStruct((N, D), jnp.float32),
)


def _p4_body(agg_a, agg_b, d, g0, x1, W2, b2, bn2g, bn2b, fcW, fcb, out_o):
    agg = (agg_a[...] + agg_b[...]) * d[...]
    t = jnp.dot(agg, W2[...], preferred_element_type=jnp.float32) + b2[...]
    g2 = jnp.maximum(t / jnp.sqrt(1.0 + EPS) * bn2g[...] + bn2b[...], 0.0) + g0[...]
    z = 0.8 * g2 + 0.2 * x1[...]
    out_o[...] = jnp.dot(z, fcW[...], preferred_element_type=jnp.float32) + fcb[...]


_p4_call = pl.pallas_call(
    _p4_body,
    grid=(GRID,),
    in_specs=[_row, _row, _row, _row, _row, _w, _b, _b, _b,
              _full((D, OUT)), _full((1, OUT))],
    out_specs=pl.BlockSpec((B1, OUT), lambda i: (i, 0)),
    out_shape=jax.ShapeDtypeStruct((N, OUT), jnp.float32),
)


def kernel(x, tW0, tb0, tln0_g, tln0_b, Wq, bq, Wk, bk, Wv, bv, tln1_g, tln1_b,
           gW0, gb0, gbn0_g, gbn0_b, W1, b1, gbn1_g, gbn1_b, W2, b2,
           gbn2_g, gbn2_b, fcW, fcb, edge_index):
    row = edge_index[0].astype(jnp.int32)
    col = edge_index[1].astype(jnp.int32)
    col3 = col.reshape(NW, NCT, CH)
    pk2 = (col | (row << 16)).reshape(NW, EPT)
    r2 = lambda v: v.reshape(1, -1)

    onesd = jnp.ones((CH, D), jnp.float32)
    zagg = jnp.zeros((RPT, D), jnp.float32)
    degp = _deg_call(col3, onesd, zagg)

    g0, M, s, sq, sk = _p1_call(
        x, tW0, r2(tb0), r2(tln0_g), r2(tln0_b), Wq, r2(bq), Wk, r2(bk),
        Wv, r2(bv), gW0, r2(gb0), r2(gbn0_g), r2(gbn0_b))

    x1, y1, dv = _p2_call(
        x, tW0, r2(tb0), r2(tln0_g), r2(tln0_b), Wq, r2(bq), Wv, r2(bv), g0,
        degp[0, :N], degp[1, :N], M, s, sq, sk, r2(tln1_g), r2(tln1_b))

    agg1 = _agg_call(y1, pk2, zagg)
    y2 = _p3_call(agg1[0, :N], agg1[1, :N], dv, g0, W1, r2(b1),
                  r2(gbn1_g), r2(gbn1_b))
    agg2 = _agg_call(y2, pk2, zagg)
    return _p4_call(agg2[0, :N], agg2[1, :N], dv, g0, x1, W2, r2(b2),
                    r2(gbn2_g), r2(gbn2_b), fcW, r2(fcb))


# speedup vs baseline: 1.1786x; 1.0064x over previous
import jax
import jax.numpy as jnp
from jax import lax
from jax.experimental import pallas as pl
from jax.experimental.pallas import tpu as pltpu
from jax.experimental.pallas import tpu_sc as plsc

N = 10000
E = 320000
D = 128
OUT = 64
EPS = 1e-5
FN = float(N)
NPAD = 10240
NC, NS = 2, 16
NW = NC * NS
RPT = NPAD // NS
EPT = E // NW
CH = 80
NCT = EPT // CH
CHA = 50
NCTA = EPT // CHA
B1 = 1000
GRID = N // B1

_mesh = plsc.VectorSubcoreMesh(core_axis_name="c", subcore_axis_name="s")


def _deg_body(col3_hbm, ones_hbm, zeros_hbm, out_hbm, colv, ones_v, deg_sh):
    c = lax.axis_index("c")
    s = lax.axis_index("s")
    wid = s * NC + c
    pltpu.sync_copy(col3_hbm.at[wid], colv)
    pltpu.sync_copy(ones_hbm, ones_v)
    pltpu.sync_copy(zeros_hbm, deg_sh.at[pl.ds(s * RPT, RPT), :])
    plsc.subcore_barrier()

    def step(k, carry):
        pltpu.sync_copy(ones_v, deg_sh.at[colv.at[k]], add=True)
        return carry

    lax.fori_loop(0, NCT, step, 0)
    plsc.subcore_barrier()
    pltpu.sync_copy(deg_sh.at[pl.ds(s * RPT, RPT), :],
                    out_hbm.at[c, pl.ds(s * RPT, RPT), :])


_deg_call = pl.kernel(
    _deg_body,
    out_type=jax.ShapeDtypeStruct((NC, NPAD, D), jnp.float32),
    mesh=_mesh,
    scratch_types=[
        pltpu.VMEM((NCT, CH), jnp.int32),
        pltpu.VMEM((CH, D), jnp.float32),
        pltpu.VMEM_SHARED((NPAD, D), jnp.float32),
    ],
)


NB = 4


def _agg_body(y_hbm, pk_hbm, zeros_hbm, out_hbm,
              pkv, colb, rowb, bufs, agg_sh, sems):
    c = lax.axis_index("c")
    s = lax.axis_index("s")
    wid = s * NC + c
    pltpu.sync_copy(pk_hbm.at[wid], pkv)

    def unpack(k, slot):
        for j in range(CHA // 16):
            v = pkv[pl.ds(k * CHA + j * 16, 16)]
            colb[slot, pl.ds(j * 16, 16)] = lax.bitwise_and(v, 0xFFFF)
            rowb[slot, pl.ds(j * 16, 16)] = lax.shift_right_logical(v, 16)

    def issue(k, slot):
        pltpu.async_copy(y_hbm.at[colb.at[slot]],
                         bufs.at[pl.ds(slot * CHA, CHA), :], sems.at[slot])

    for k in range(NB - 1):
        unpack(k, k)
        issue(k, k)
    pltpu.sync_copy(zeros_hbm, agg_sh.at[pl.ds(s * RPT, RPT), :])
    plsc.subcore_barrier()

    def step(k, carry):
        slot = lax.rem(k, NB)
        nk = k + NB - 1
        nslot = lax.rem(nk, NB)

        @pl.when(nk < NCTA)
        def _():
            unpack(nk, nslot)
            issue(nk, nslot)

        pltpu.make_async_copy(y_hbm.at[colb.at[slot]],
                              bufs.at[pl.ds(slot * CHA, CHA), :],
                              sems.at[slot]).wait()
        pltpu.sync_copy(bufs.at[pl.ds(slot * CHA, CHA), :],
                        agg_sh.at[rowb.at[slot]], add=True)
        return carry

    lax.fori_loop(0, NCTA, step, 0)
    plsc.subcore_barrier()
    pltpu.sync_copy(agg_sh.at[pl.ds(s * RPT, RPT), :],
                    out_hbm.at[c, pl.ds(s * RPT, RPT), :])


_agg_call = pl.kernel(
    _agg_body,
    out_type=jax.ShapeDtypeStruct((NC, NPAD, D), jnp.float32),
    mesh=_mesh,
    scratch_types=[
        pltpu.VMEM((EPT,), jnp.int32),
        pltpu.VMEM((NB, CHA), jnp.int32),
        pltpu.VMEM((NB, CHA), jnp.int32),
        pltpu.VMEM((NB * CHA, D), jnp.float32),
        pltpu.VMEM_SHARED((NPAD, D), jnp.float32),
        pltpu.SemaphoreType.DMA((NB,)),
    ],
)


def _p1_body(x_ref, tW0, tb0, ln0g, ln0b, Wq, bq, Wk, bk, Wv, bv,
             gW0, gb0, bn0g, bn0b,
             g0_o, M_o, s_o, sq_o, sk_o):
    i = pl.program_id(0)
    x = x_ref[...]
    t = jnp.dot(x, tW0[...], preferred_element_type=jnp.float32) + tb0[...]
    mu = jnp.mean(t, axis=-1, keepdims=True)
    var = jnp.mean((t - mu) ** 2, axis=-1, keepdims=True)
    h = jnp.maximum((t - mu) / jnp.sqrt(var + EPS) * ln0g[...] + ln0b[...], 0.0)
    qs = jnp.dot(h, Wq[...], preferred_element_type=jnp.float32) + bq[...]
    ks = jnp.dot(h, Wk[...], preferred_element_type=jnp.float32) + bk[...]
    vs = jnp.dot(h, Wv[...], preferred_element_type=jnp.float32) + bv[...]
    g0 = jnp.dot(x, gW0[...], preferred_element_type=jnp.float32) + gb0[...]
    g0_o[...] = jnp.maximum(g0 / jnp.sqrt(1.0 + EPS) * bn0g[...] + bn0b[...], 0.0)

    @pl.when(i == 0)
    def _():
        M_o[...] = jnp.zeros_like(M_o)
        s_o[...] = jnp.zeros_like(s_o)
        sq_o[...] = jnp.zeros_like(sq_o)
        sk_o[...] = jnp.zeros_like(sk_o)

    M_o[...] += lax.dot_general(ks, vs, (((0,), (0,)), ((), ())),
                                preferred_element_type=jnp.float32)
    s_o[...] += jnp.sum(ks, axis=0, keepdims=True)
    sq_o[...] += jnp.sum(qs * qs)
    sk_o[...] += jnp.sum(ks * ks)


def _full(shp):
    return pl.BlockSpec(shp, lambda *_: tuple(0 for _ in shp))


_row = pl.BlockSpec((B1, D), lambda i: (i, 0))
_row1 = pl.BlockSpec((B1, 1), lambda i: (i, 0))
_w = _full((D, D))
_b = _full((1, D))

_p1_call = pl.pallas_call(
    _p1_body,
    grid=(GRID,),
    in_specs=[_row, _w, _b, _b, _b, _w, _b, _w, _b, _w, _b, _w, _b, _b, _b],
    out_specs=[_row, _w, _b, _full((1, 1)), _full((1, 1))],
    out_shape=[
        jax.ShapeDtypeStruct((N, D), jnp.float32),
        jax.ShapeDtypeStruct((D, D), jnp.float32),
        jax.ShapeDtypeStruct((1, D), jnp.float32),
        jax.ShapeDtypeStruct((1, 1), jnp.float32),
        jax.ShapeDtypeStruct((1, 1), jnp.float32),
    ],
)


def _p2_body(x_ref, tW0, tb0, ln0g, ln0b, Wq, bq, Wv, bv, g0,
             dega, degb, M, s, sq, sk, ln1g, ln1b,
             x1_o, y1_o, d_o):
    x = x_ref[...]
    tt = jnp.dot(x, tW0[...], preferred_element_type=jnp.float32) + tb0[...]
    mu0 = jnp.mean(tt, axis=-1, keepdims=True)
    var0 = jnp.mean((tt - mu0) ** 2, axis=-1, keepdims=True)
    h = jnp.maximum(
        (tt - mu0) / jnp.sqrt(var0 + EPS) * ln0g[...] + ln0b[...], 0.0)
    q = jnp.dot(h, Wq[...], preferred_element_type=jnp.float32) + bq[...]
    vs = jnp.dot(h, Wv[...], preferred_element_type=jnp.float32) + bv[...]
    cc = lax.rsqrt(sq[0, 0] * sk[0, 0])
    num = jnp.dot(q, M[...], preferred_element_type=jnp.float32) * cc + FN * vs
    den = lax.dot_general(q, s[...], (((1,), (1,)), ((), ())),
                          preferred_element_type=jnp.float32) * cc + FN
    t = (num / den + h) * 0.5
    mu = jnp.mean(t, axis=-1, keepdims=True)
    var = jnp.mean((t - mu) ** 2, axis=-1, keepdims=True)
    x1_o[...] = jnp.maximum(
        (t - mu) / jnp.sqrt(var + EPS) * ln1g[...] + ln1b[...], 0.0)
    degsum = dega[...] + degb[...]
    dv = jnp.where(degsum > 0.0, lax.rsqrt(degsum), 0.0)
    d_o[...] = dv
    y1_o[...] = dv * g0[...]


_p2_call = pl.pallas_call(
    _p2_body,
    grid=(GRID,),
    in_specs=[_row, _w, _b, _b, _b, _w, _b, _w, _b, _row, _row, _row,
              _w, _b, _full((1, 1)), _full((1, 1)), _b, _b],
    out_specs=[_row, _row, _row],
    out_shape=[
        jax.ShapeDtypeStruct((N, D), jnp.float32),
        jax.ShapeDtypeStruct((N, D), jnp.float32),
        jax.ShapeDtypeStruct((N, D), jnp.float32),
    ],
)


def _p3_body(agg_a, agg_b, d, g0, W1, b1, bn1g, bn1b, y2_o):
    agg = (agg_a[...] + agg_b[...]) * d[...]
    t = jnp.dot(agg, W1[...], preferred_element_type=jnp.float32) + b1[...]
    g1 = jnp.maximum(t / jnp.sqrt(1.0 + EPS) * bn1g[...] + bn1b[...], 0.0) + g0[...]
    y2_o[...] = d[...] * g1


_p3_call = pl.pallas_call(
    _p3_body,
    grid=(GRID,),
    in_specs=[_row, _row, _row, _row, _w, _b, _b, _b],
    out_specs=_row,
    out_shape=jax.ShapeDtypeStruct((N, D), jnp.float32),
)


def _p4_body(agg_a, agg_b, d, g0, x1, W2, b2, bn2g, bn2b, fcW, fcb, out_o):
    agg = (agg_a[...] + agg_b[...]) * d[...]
    t = jnp.dot(agg, W2[...], preferred_element_type=jnp.float32) + b2[...]
    g2 = jnp.maximum(t / jnp.sqrt(1.0 + EPS) * bn2g[...] + bn2b[...], 0.0) + g0[...]
    z = 0.8 * g2 + 0.2 * x1[...]
    out_o[...] = jnp.dot(z, fcW[...], preferred_element_type=jnp.float32) + fcb[...]


_p4_call = pl.pallas_call(
    _p4_body,
    grid=(GRID,),
    in_specs=[_row, _row, _row, _row, _row, _w, _b, _b, _b,
              _full((D, OUT)), _full((1, OUT))],
    out_specs=pl.BlockSpec((B1, OUT), lambda i: (i, 0)),
    out_shape=jax.ShapeDtypeStruct((N, OUT), jnp.float32),
)


def kernel(x, tW0, tb0, tln0_g, tln0_b, Wq, bq, Wk, bk, Wv, bv, tln1_g, tln1_b,
           gW0, gb0, gbn0_g, gbn0_b, W1, b1, gbn1_g, gbn1_b, W2, b2,
           gbn2_g, gbn2_b, fcW, fcb, edge_index):
    row = edge_index[0].astype(jnp.int32)
    col = edge_index[1].astype(jnp.int32)
    col3 = col.reshape(NW, NCT, CH)
    pk2 = (col | (row << 16)).reshape(NW, EPT)
    r2 = lambda v: v.reshape(1, -1)

    onesd = jnp.ones((CH, D), jnp.float32)
    zagg = jnp.zeros((RPT, D), jnp.float32)
    degp = _deg_call(col3, onesd, zagg)

    g0, M, s, sq, sk = _p1_call(
        x, tW0, r2(tb0), r2(tln0_g), r2(tln0_b), Wq, r2(bq), Wk, r2(bk),
        Wv, r2(bv), gW0, r2(gb0), r2(gbn0_g), r2(gbn0_b))

    x1, y1, dv = _p2_call(
        x, tW0, r2(tb0), r2(tln0_g), r2(tln0_b), Wq, r2(bq), Wv, r2(bv), g0,
        degp[0, :N], degp[1, :N], M, s, sq, sk, r2(tln1_g), r2(tln1_b))

    agg1 = _agg_call(y1, pk2, zagg)
    y2 = _p3_call(agg1[0, :N], agg1[1, :N], dv, g0, W1, r2(b1),
                  r2(gbn1_g), r2(gbn1_b))
    agg2 = _agg_call(y2, pk2, zagg)
    return _p4_call(agg2[0, :N], agg2[1, :N], dv, g0, x1, W2, r2(b2),
                    r2(gbn2_g), r2(gbn2_b), fcW, r2(fcb))
